# baseline (speedup 1.0000x reference)
import dataclasses
import functools

import jax
import jax.numpy as jnp
from jax.experimental import pallas as pl
from jax.experimental.pallas import tpu as pltpu
from jax.experimental.pallas import tpu_sc as plsc

B = 16384
ED = 16
NF = 13
NC, NB, CD = 26, 20, 8
NCLS = NC * NB
H1, H2 = 256, 128
PACK = 128 // ED

GATHER_W = 128
MLP_BLK = 512


REPACK_W = 1024
UA_BLKS = 640
UA_ROWS = UA_BLKS * REPACK_W // PACK
TCB_W = 16384
UB_START_BLK = UA_BLKS * REPACK_W // TCB_W


def _sc_repack(user_tT, diner_tT):
    mesh = plsc.VectorSubcoreMesh(core_axis_name="c", subcore_axis_name="s")
    nu = user_tT.shape[1]
    nd = diner_tT.shape[1]
    cp = pltpu.CompilerParams()
    if "needs_layout_passes" in pltpu.CompilerParams.__dataclass_fields__:
        cp = dataclasses.replace(cp, needs_layout_passes=False)

    @functools.partial(
        pl.kernel,
        out_type=(
            jax.ShapeDtypeStruct((UA_ROWS, 128), jnp.float32),
            jax.ShapeDtypeStruct((nd // PACK, 128), jnp.float32),
        ),
        mesh=mesh,
        compiler_params=cp,
    )
    def repack_kernel(ut_hbm, dt_hbm, up_hbm, dp_hbm):
        def body(in_vmem, out_vmem):
            d_vec = jax.lax.iota(jnp.int32, ED)

            @plsc.parallel_loop(0, REPACK_W // PACK)
            def _(p):
                base = jnp.full((ED,), p * PACK, jnp.int32)
                row = jnp.full((ED,), p, jnp.int32)
                for j in range(PACK):
                    v = plsc.load_gather(in_vmem, [d_vec, base + j])
                    plsc.store_scatter(out_vmem, [row, d_vec + j * ED], v)

        def run(t_hbm, out_hbm, nblk):
            pltpu.emit_pipeline(
                body,
                grid=(nblk,),
                in_specs=[pl.BlockSpec((ED, REPACK_W), lambda i: (0, i))],
                out_specs=[pl.BlockSpec((REPACK_W // PACK, 128),
                                        lambda i: (i, 0))],
                core_axis_name=("c", "s"),
                dimension_semantics=(pltpu.PARALLEL,),
            )(t_hbm, out_hbm)

        run(ut_hbm, up_hbm, UA_BLKS)
        run(dt_hbm, dp_hbm, nd // REPACK_W)

    return repack_kernel(user_tT, diner_tT)


def _tc_repack_partB(tT):
    n = tT.shape[1]
    rows_b = (n - UA_BLKS * REPACK_W) // PACK
    grid = ((n - UA_BLKS * REPACK_W + TCB_W - 1) // TCB_W,)
    return pl.pallas_call(
        _repack_body,
        grid=grid,
        in_specs=[pl.BlockSpec((ED, TCB_W), lambda i: (0, i + UB_START_BLK))],
        out_specs=pl.BlockSpec((TCB_W // PACK, 128), lambda i: (i, 0)),
        out_shape=jax.ShapeDtypeStruct((rows_b, 128), jnp.float32),
    )(tT)


def _repack_body(in_ref, out_ref):
    x = in_ref[...]
    y = jnp.transpose(x)
    y3 = y.reshape(-1, PACK, ED)
    out_ref[...] = jnp.concatenate([y3[:, j, :] for j in range(PACK)],
                                   axis=1)


def _tail_body(t_ref, packed_ref, out_ref):
    del packed_ref
    x = t_ref[...]
    y = jnp.transpose(x)
    y3 = y.reshape(-1, PACK, ED)
    out_ref[...] = jnp.concatenate([y3[:, j, :] for j in range(PACK)],
                                   axis=1)


def _tc_tail_repack(tT, packed):
    n = tT.shape[1]
    k = n // REPACK_W
    rows = packed.shape[0]
    return pl.pallas_call(
        _tail_body,
        grid=(1,),
        in_specs=[
            pl.BlockSpec((ED, REPACK_W), lambda i: (0, k)),
            pl.BlockSpec(memory_space=pltpu.MemorySpace.HBM),
        ],
        out_specs=pl.BlockSpec((REPACK_W // PACK, 128), lambda i: (k, 0)),
        out_shape=jax.ShapeDtypeStruct((rows, 128), jnp.float32),
        input_output_aliases={1: 0},
    )(tT, packed)


def _sc_gather(tab_a, idx_a, tab_b, idx_b, tab_d, idx_d):
    mesh = plsc.VectorSubcoreMesh(core_axis_name="c", subcore_axis_name="s")

    @functools.partial(
        pl.kernel,
        out_type=(
            jax.ShapeDtypeStruct((B, 128), jnp.float32),
            jax.ShapeDtypeStruct((B, 128), jnp.float32),
            jax.ShapeDtypeStruct((B, 128), jnp.float32),
        ),
        mesh=mesh,
    )
    def gather_kernel(ta_hbm, ia_hbm, tb_hbm, ib_hbm, td_hbm, id_hbm,
                      oa_hbm, ob_hbm, od_hbm):
        def make_body(table_hbm):
            def body(i_vmem, o_vmem):
                pltpu.sync_copy(table_hbm.at[i_vmem.at[0]], o_vmem)
            return body

        def run(table_hbm, idx_hbm, out_hbm):
            pltpu.emit_pipeline(
                make_body(table_hbm),
                grid=(B // GATHER_W,),
                in_specs=[pl.BlockSpec((1, GATHER_W), lambda i: (0, i))],
                out_specs=[pl.BlockSpec((GATHER_W, 128), lambda i: (i, 0))],
                core_axis_name=("c", "s"),
                dimension_semantics=(pltpu.PARALLEL,),
            )(idx_hbm, out_hbm)

        run(ta_hbm, ia_hbm, oa_hbm)
        run(tb_hbm, ib_hbm, ob_hbm)
        run(td_hbm, id_hbm, od_hbm)

    return gather_kernel(tab_a, idx_a, tab_b, idx_b, tab_d, idx_d)


def _mlp_body(uepa, uepb, dep, mods, f, bkt, kpat, expand, patt, Wcat,
              Wu, Wd, Wf, b1, fn_g, fn_b, g1, bb1, W2, b2, g2, bb2,
              W3, b3, out):
    mc = jnp.transpose(mods[...])
    umod, dmod, sel_a = mc[:, 0:1], mc[:, 1:2], mc[:, 2:3]
    mu = (umod == kpat[...]).astype(jnp.bfloat16)
    md = (dmod == kpat[...]).astype(jnp.bfloat16)
    ue_pack = uepa[...] * sel_a + uepb[...] * (1.0 - sel_a)
    pu = ue_pack.astype(jnp.bfloat16) * mu
    pd = dep[...].astype(jnp.bfloat16) * md

    fx = f[...]
    m = jnp.mean(fx, axis=-1, keepdims=True)
    v = jnp.mean((fx - m) ** 2, axis=-1, keepdims=True)
    fln = (fx - m) * jax.lax.rsqrt(v + 1e-5) * fn_g[...] + fn_b[...]

    rep = jnp.dot(bkt[...], expand[...], preferred_element_type=jnp.float32)
    mh = (rep == patt[...]).astype(jnp.bfloat16)

    h = jnp.dot(mh, Wcat[...], preferred_element_type=jnp.float32)
    h = h + jnp.dot(pu, Wu[...], preferred_element_type=jnp.float32)
    h = h + jnp.dot(pd, Wd[...], preferred_element_type=jnp.float32)
    h = h + jnp.dot(fln.astype(jnp.bfloat16), Wf[...],
                    preferred_element_type=jnp.float32)
    h = h + b1[...]
    m = jnp.mean(h, axis=-1, keepdims=True)
    v = jnp.mean((h - m) ** 2, axis=-1, keepdims=True)
    h = (h - m) * jax.lax.rsqrt(v + 1e-5) * g1[...] + bb1[...]
    h = jnp.maximum(h, 0.0).astype(jnp.bfloat16)

    h = jnp.dot(h, W2[...], preferred_element_type=jnp.float32) + b2[...]
    m = jnp.mean(h, axis=-1, keepdims=True)
    v = jnp.mean((h - m) ** 2, axis=-1, keepdims=True)
    h = (h - m) * jax.lax.rsqrt(v + 1e-5) * g2[...] + bb2[...]
    h = jnp.maximum(h, 0.0).astype(jnp.bfloat16)

    o = jnp.dot(h, W3[...], preferred_element_type=jnp.float32) + b3[...]
    out[...] = jax.nn.sigmoid(o)


def _tc_mlp(uepa, uepb, dep, mods, features, bkt, kpat, expand, patt, Wcat,
            Wu, Wd, Wf, b1, fn_g, fn_b, ln1_g, ln1_b, W2, b2, ln2_g, ln2_b,
            W3, b3):
    grid = (B // MLP_BLK,)

    def row_spec(cols):
        return pl.BlockSpec((MLP_BLK, cols), lambda i: (i, 0))

    def full_spec(a):
        return pl.BlockSpec(a.shape, lambda i: (0,) * a.ndim)

    out = pl.pallas_call(
        _mlp_body,
        grid=grid,
        in_specs=[
            row_spec(128), row_spec(128), row_spec(128),
            pl.BlockSpec((3, MLP_BLK), lambda i: (0, i)),
            row_spec(NF), row_spec(NC),
            full_spec(kpat), full_spec(expand), full_spec(patt),
            full_spec(Wcat), full_spec(Wu), full_spec(Wd), full_spec(Wf),
            full_spec(b1), full_spec(fn_g), full_spec(fn_b),
            full_spec(ln1_g), full_spec(ln1_b),
            full_spec(W2), full_spec(b2), full_spec(ln2_g), full_spec(ln2_b),
            full_spec(W3), full_spec(b3),
        ],
        out_specs=pl.BlockSpec((MLP_BLK, 1), lambda i: (i, 0)),
        out_shape=jax.ShapeDtypeStruct((B, 1), jnp.float32),
    )(uepa, uepb, dep, mods, features, bkt, kpat, expand, patt, Wcat,
      Wu, Wd, Wf, b1, fn_g, fn_b, ln1_g, ln1_b, W2, b2, ln2_g, ln2_b,
      W3, b3)
    return out[:, 0]


def kernel(user_idx, diner_idx, features, categorical_bucket_idx,
           user_table, diner_table, cat_tables, fn_g, fn_b, W1, b1,
           ln1_g, ln1_b, W2, b2, ln2_g, ln2_b, W3, b3):
    uidx = user_idx.astype(jnp.int32)
    didx = diner_idx.astype(jnp.int32)

    upack_a, diner_packed = _sc_repack(user_table.T, diner_table.T)
    upack_b = _tc_repack_partB(user_table.T)
    diner_packed = _tc_tail_repack(diner_table.T, diner_packed)

    upk = uidx // PACK
    upk_a = jnp.minimum(upk, UA_ROWS - 1)
    upk_b = jnp.clip(upk - UA_ROWS, 0, upack_b.shape[0] - 1)
    uepa, uepb, dep = _sc_gather(upack_a, upk_a.reshape(1, B),
                                 upack_b, upk_b.reshape(1, B),
                                 diner_packed, (didx // PACK).reshape(1, B))
    mods = jnp.stack([
        (uidx % PACK).astype(jnp.float32),
        (didx % PACK).astype(jnp.float32),
        (upk < UA_ROWS).astype(jnp.float32),
    ])

    W1c = W1[2 * ED + NF:].reshape(NC, CD, H1)
    Wcat = jnp.einsum("cbd,cdh->cbh", cat_tables, W1c,
                      preferred_element_type=jnp.float32)
    Wcat = Wcat.reshape(NCLS, H1).astype(jnp.bfloat16)

    cls = jnp.arange(NCLS, dtype=jnp.int32)
    expand = (cls[None, :] // NB == jnp.arange(NC, dtype=jnp.int32)[:, None])
    expand = expand.astype(jnp.bfloat16)
    patt = (cls % NB).astype(jnp.float32).reshape(1, NCLS)
    bkt = categorical_bucket_idx.astype(jnp.bfloat16)
    kpat = (jnp.arange(128, dtype=jnp.int32) // ED).astype(
        jnp.float32).reshape(1, 128)

    Wb = W1.astype(jnp.bfloat16)
    Wu = jnp.tile(Wb[:ED], (PACK, 1))
    Wd = jnp.tile(Wb[ED:2 * ED], (PACK, 1))
    Wf = Wb[2 * ED:2 * ED + NF]

    out = _tc_mlp(uepa, uepb, dep, mods,
                  features, bkt, kpat, expand, patt, Wcat, Wu, Wd, Wf,
                  b1.reshape(1, H1),
                  fn_g.reshape(1, NF), fn_b.reshape(1, NF),
                  ln1_g.reshape(1, H1), ln1_b.reshape(1, H1),
                  W2.astype(jnp.bfloat16), b2.reshape(1, H2),
                  ln2_g.reshape(1, H2), ln2_b.reshape(1, H2),
                  W3.astype(jnp.bfloat16), b3.reshape(1, 1))
    return out


# speedup vs baseline: 3.2174x; 3.2174x over previous
import dataclasses
import functools

import jax
import jax.numpy as jnp
from jax.experimental import pallas as pl
from jax.experimental.pallas import tpu as pltpu
from jax.experimental.pallas import tpu_sc as plsc

B = 16384
ED = 16
NF = 13
NC, NB, CD = 26, 20, 8
NCLS = NC * NB
H1, H2 = 256, 128
PACK = 128 // ED

GATHER_W = 128
MLP_BLK = 512


REPACK_W = 1024
UA_BLKS = 640
UA_ROWS = UA_BLKS * REPACK_W // PACK
TCB_W = 16384
UB_START_BLK = UA_BLKS * REPACK_W // TCB_W


def _sc_repack(user_tT, diner_tT):
    mesh = plsc.VectorSubcoreMesh(core_axis_name="c", subcore_axis_name="s")
    nu = user_tT.shape[1]
    nd = diner_tT.shape[1]
    cp = pltpu.CompilerParams()
    if "needs_layout_passes" in pltpu.CompilerParams.__dataclass_fields__:
        cp = dataclasses.replace(cp, needs_layout_passes=False)

    @functools.partial(
        pl.kernel,
        out_type=(
            jax.ShapeDtypeStruct((UA_ROWS, 128), jnp.float32),
            jax.ShapeDtypeStruct((nd // PACK, 128), jnp.float32),
        ),
        mesh=mesh,
        compiler_params=cp,
    )
    def repack_kernel(ut_hbm, dt_hbm, up_hbm, dp_hbm):
        def body(in_vmem, out_vmem):
            d_vec = jax.lax.iota(jnp.int32, ED)

            @plsc.parallel_loop(0, REPACK_W // PACK)
            def _(p):
                base = jnp.full((ED,), p * PACK, jnp.int32)
                row = jnp.full((ED,), p, jnp.int32)
                for j in range(PACK):
                    v = plsc.load_gather(in_vmem, [d_vec, base + j])
                    plsc.store_scatter(out_vmem, [row, d_vec + j * ED], v)

        def run(t_hbm, out_hbm, nblk):
            pltpu.emit_pipeline(
                body,
                grid=(nblk,),
                in_specs=[pl.BlockSpec((ED, REPACK_W), lambda i: (0, i))],
                out_specs=[pl.BlockSpec((REPACK_W // PACK, 128),
                                        lambda i: (i, 0))],
                core_axis_name=("c", "s"),
                dimension_semantics=(pltpu.PARALLEL,),
            )(t_hbm, out_hbm)

        run(ut_hbm, up_hbm, UA_BLKS)
        run(dt_hbm, dp_hbm, nd // REPACK_W)

    return repack_kernel(user_tT, diner_tT)


def _tc_repack_partB(tT):
    n = tT.shape[1]
    rows_b = (n - UA_BLKS * REPACK_W) // PACK
    grid = ((n - UA_BLKS * REPACK_W + TCB_W - 1) // TCB_W,)
    return pl.pallas_call(
        _repack_body,
        grid=grid,
        in_specs=[pl.BlockSpec((ED, TCB_W), lambda i: (0, i + UB_START_BLK))],
        out_specs=pl.BlockSpec((TCB_W // PACK, 128), lambda i: (i, 0)),
        out_shape=jax.ShapeDtypeStruct((rows_b, 128), jnp.float32),
    )(tT)


def _repack_body(in_ref, out_ref):
    x = in_ref[...]
    y = jnp.transpose(x)
    y3 = y.reshape(-1, PACK, ED)
    out_ref[...] = jnp.concatenate([y3[:, j, :] for j in range(PACK)],
                                   axis=1)


def _tail_body(t_ref, packed_ref, out_ref):
    del packed_ref
    x = t_ref[...]
    y = jnp.transpose(x)
    y3 = y.reshape(-1, PACK, ED)
    out_ref[...] = jnp.concatenate([y3[:, j, :] for j in range(PACK)],
                                   axis=1)


def _tc_tail_repack(tT, packed):
    n = tT.shape[1]
    k = n // REPACK_W
    rows = packed.shape[0]
    return pl.pallas_call(
        _tail_body,
        grid=(1,),
        in_specs=[
            pl.BlockSpec((ED, REPACK_W), lambda i: (0, k)),
            pl.BlockSpec(memory_space=pltpu.MemorySpace.HBM),
        ],
        out_specs=pl.BlockSpec((REPACK_W // PACK, 128), lambda i: (k, 0)),
        out_shape=jax.ShapeDtypeStruct((rows, 128), jnp.float32),
        input_output_aliases={1: 0},
    )(tT, packed)


def _sc_gather(tab_a, idx_a, tab_b, idx_b, tab_d, idx_d):
    mesh = plsc.VectorSubcoreMesh(core_axis_name="c", subcore_axis_name="s")

    @functools.partial(
        pl.kernel,
        out_type=(
            jax.ShapeDtypeStruct((B, 128), jnp.float32),
            jax.ShapeDtypeStruct((B, 128), jnp.float32),
            jax.ShapeDtypeStruct((B, 128), jnp.float32),
        ),
        mesh=mesh,
    )
    def gather_kernel(ta_hbm, ia_hbm, tb_hbm, ib_hbm, td_hbm, id_hbm,
                      oa_hbm, ob_hbm, od_hbm):
        def make_body(table_hbm):
            def body(i_vmem, o_vmem):
                pltpu.sync_copy(table_hbm.at[i_vmem.at[0]], o_vmem)
            return body

        def run(table_hbm, idx_hbm, out_hbm):
            pltpu.emit_pipeline(
                make_body(table_hbm),
                grid=(B // GATHER_W,),
                in_specs=[pl.BlockSpec((1, GATHER_W), lambda i: (0, i))],
                out_specs=[pl.BlockSpec((GATHER_W, 128), lambda i: (i, 0))],
                core_axis_name=("c", "s"),
                dimension_semantics=(pltpu.PARALLEL,),
            )(idx_hbm, out_hbm)

        run(ta_hbm, ia_hbm, oa_hbm)
        run(tb_hbm, ib_hbm, ob_hbm)
        run(td_hbm, id_hbm, od_hbm)

    return gather_kernel(tab_a, idx_a, tab_b, idx_b, tab_d, idx_d)


def _mlp_body(uepa, uepb, dep, mods, f, bkt, kpat, expand, patt, Wcat,
              Wu, Wd, Wf, b1, fn_g, fn_b, g1, bb1, W2, b2, g2, bb2,
              W3, b3, out):
    mc = jnp.transpose(mods[...])
    umod, dmod, sel_a = mc[:, 0:1], mc[:, 1:2], mc[:, 2:3]
    mu = (umod == kpat[...]).astype(jnp.bfloat16)
    md = (dmod == kpat[...]).astype(jnp.bfloat16)
    ue_pack = uepa[...] * sel_a + uepb[...] * (1.0 - sel_a)
    pu = ue_pack.astype(jnp.bfloat16) * mu
    pd = dep[...].astype(jnp.bfloat16) * md

    fx = f[...]
    m = jnp.mean(fx, axis=-1, keepdims=True)
    v = jnp.mean((fx - m) ** 2, axis=-1, keepdims=True)
    fln = (fx - m) * jax.lax.rsqrt(v + 1e-5) * fn_g[...] + fn_b[...]

    rep = jnp.dot(bkt[...], expand[...], preferred_element_type=jnp.float32)
    mh = (rep == patt[...]).astype(jnp.bfloat16)

    h = jnp.dot(mh, Wcat[...], preferred_element_type=jnp.float32)
    h = h + jnp.dot(pu, Wu[...], preferred_element_type=jnp.float32)
    h = h + jnp.dot(pd, Wd[...], preferred_element_type=jnp.float32)
    h = h + jnp.dot(fln.astype(jnp.bfloat16), Wf[...],
                    preferred_element_type=jnp.float32)
    h = h + b1[...]
    m = jnp.mean(h, axis=-1, keepdims=True)
    v = jnp.mean((h - m) ** 2, axis=-1, keepdims=True)
    h = (h - m) * jax.lax.rsqrt(v + 1e-5) * g1[...] + bb1[...]
    h = jnp.maximum(h, 0.0).astype(jnp.bfloat16)

    h = jnp.dot(h, W2[...], preferred_element_type=jnp.float32) + b2[...]
    m = jnp.mean(h, axis=-1, keepdims=True)
    v = jnp.mean((h - m) ** 2, axis=-1, keepdims=True)
    h = (h - m) * jax.lax.rsqrt(v + 1e-5) * g2[...] + bb2[...]
    h = jnp.maximum(h, 0.0).astype(jnp.bfloat16)

    o = jnp.dot(h, W3[...], preferred_element_type=jnp.float32) + b3[...]
    out[...] = jax.nn.sigmoid(o)


def _tc_mlp(uepa, uepb, dep, mods, features, bkt, kpat, expand, patt, Wcat,
            Wu, Wd, Wf, b1, fn_g, fn_b, ln1_g, ln1_b, W2, b2, ln2_g, ln2_b,
            W3, b3):
    grid = (B // MLP_BLK,)

    def row_spec(cols):
        return pl.BlockSpec((MLP_BLK, cols), lambda i: (i, 0))

    def full_spec(a):
        return pl.BlockSpec(a.shape, lambda i: (0,) * a.ndim)

    out = pl.pallas_call(
        _mlp_body,
        grid=grid,
        in_specs=[
            row_spec(128), row_spec(128), row_spec(128),
            pl.BlockSpec((3, MLP_BLK), lambda i: (0, i)),
            row_spec(NF), row_spec(NC),
            full_spec(kpat), full_spec(expand), full_spec(patt),
            full_spec(Wcat), full_spec(Wu), full_spec(Wd), full_spec(Wf),
            full_spec(b1), full_spec(fn_g), full_spec(fn_b),
            full_spec(ln1_g), full_spec(ln1_b),
            full_spec(W2), full_spec(b2), full_spec(ln2_g), full_spec(ln2_b),
            full_spec(W3), full_spec(b3),
        ],
        out_specs=pl.BlockSpec((MLP_BLK, 1), lambda i: (i, 0)),
        out_shape=jax.ShapeDtypeStruct((B, 1), jnp.float32),
    )(uepa, uepb, dep, mods, features, bkt, kpat, expand, patt, Wcat,
      Wu, Wd, Wf, b1, fn_g, fn_b, ln1_g, ln1_b, W2, b2, ln2_g, ln2_b,
      W3, b3)
    return out[:, 0]


def kernel(user_idx, diner_idx, features, categorical_bucket_idx,
           user_table, diner_table, cat_tables, fn_g, fn_b, W1, b1,
           ln1_g, ln1_b, W2, b2, ln2_g, ln2_b, W3, b3):
    uidx = user_idx.astype(jnp.int32)
    didx = diner_idx.astype(jnp.int32)

    upack_a, diner_packed = _sc_repack(user_table.T, diner_table.T)
    upack_b = _tc_repack_partB(user_table.T)
    diner_packed = _tc_tail_repack(diner_table.T, diner_packed)

    upk = uidx // PACK
    in_a = upk < UA_ROWS
    upk_a = jnp.where(in_a, upk, upk % UA_ROWS)
    upk_b = jnp.where(in_a, upk % jnp.int32(upack_b.shape[0]), upk - UA_ROWS)
    uepa, uepb, dep = _sc_gather(upack_a, upk_a.reshape(1, B),
                                 upack_b, upk_b.reshape(1, B),
                                 diner_packed, (didx // PACK).reshape(1, B))
    mods = jnp.stack([
        (uidx % PACK).astype(jnp.float32),
        (didx % PACK).astype(jnp.float32),
        (upk < UA_ROWS).astype(jnp.float32),
    ])

    W1c = W1[2 * ED + NF:].reshape(NC, CD, H1)
    Wcat = jnp.einsum("cbd,cdh->cbh", cat_tables, W1c,
                      preferred_element_type=jnp.float32)
    Wcat = Wcat.reshape(NCLS, H1).astype(jnp.bfloat16)

    cls = jnp.arange(NCLS, dtype=jnp.int32)
    expand = (cls[None, :] // NB == jnp.arange(NC, dtype=jnp.int32)[:, None])
    expand = expand.astype(jnp.bfloat16)
    patt = (cls % NB).astype(jnp.float32).reshape(1, NCLS)
    bkt = categorical_bucket_idx.astype(jnp.bfloat16)
    kpat = (jnp.arange(128, dtype=jnp.int32) // ED).astype(
        jnp.float32).reshape(1, 128)

    Wb = W1.astype(jnp.bfloat16)
    Wu = jnp.tile(Wb[:ED], (PACK, 1))
    Wd = jnp.tile(Wb[ED:2 * ED], (PACK, 1))
    Wf = Wb[2 * ED:2 * ED + NF]

    out = _tc_mlp(uepa, uepb, dep, mods,
                  features, bkt, kpat, expand, patt, Wcat, Wu, Wd, Wf,
                  b1.reshape(1, H1),
                  fn_g.reshape(1, NF), fn_b.reshape(1, NF),
                  ln1_g.reshape(1, H1), ln1_b.reshape(1, H1),
                  W2.astype(jnp.bfloat16), b2.reshape(1, H2),
                  ln2_g.reshape(1, H2), ln2_b.reshape(1, H2),
                  W3.astype(jnp.bfloat16), b3.reshape(1, 1))
    return out


# speedup vs baseline: 3.4621x; 1.0761x over previous
import dataclasses
import functools

import jax
import jax.numpy as jnp
from jax.experimental import pallas as pl
from jax.experimental.pallas import tpu as pltpu
from jax.experimental.pallas import tpu_sc as plsc

B = 16384
ED = 16
NF = 13
NC, NB, CD = 26, 20, 8
NCLS = NC * NB
H1, H2 = 256, 128
PACK = 128 // ED

GATHER_W = 128
MLP_BLK = 512


REPACK_W = 1024
UA_BLKS = 544
UA_ROWS = UA_BLKS * REPACK_W // PACK
TCB_W = 16384
UB_START_BLK = UA_BLKS * REPACK_W // TCB_W


def _sc_repack(user_tT, diner_tT):
    mesh = plsc.VectorSubcoreMesh(core_axis_name="c", subcore_axis_name="s")
    nu = user_tT.shape[1]
    nd = diner_tT.shape[1]
    cp = pltpu.CompilerParams()
    if "needs_layout_passes" in pltpu.CompilerParams.__dataclass_fields__:
        cp = dataclasses.replace(cp, needs_layout_passes=False)

    @functools.partial(
        pl.kernel,
        out_type=(
            jax.ShapeDtypeStruct((UA_ROWS, 128), jnp.float32),
            jax.ShapeDtypeStruct((nd // PACK, 128), jnp.float32),
        ),
        mesh=mesh,
        compiler_params=cp,
    )
    def repack_kernel(ut_hbm, dt_hbm, up_hbm, dp_hbm):
        def body(in_vmem, out_vmem):
            d_vec = jax.lax.iota(jnp.int32, ED)

            @plsc.parallel_loop(0, REPACK_W // PACK)
            def _(p):
                base = jnp.full((ED,), p * PACK, jnp.int32)
                row = jnp.full((ED,), p, jnp.int32)
                for j in range(PACK):
                    v = plsc.load_gather(in_vmem, [d_vec, base + j])
                    plsc.store_scatter(out_vmem, [row, d_vec + j * ED], v)

        def run(t_hbm, out_hbm, nblk):
            pltpu.emit_pipeline(
                body,
                grid=(nblk,),
                in_specs=[pl.BlockSpec((ED, REPACK_W), lambda i: (0, i))],
                out_specs=[pl.BlockSpec((REPACK_W // PACK, 128),
                                        lambda i: (i, 0))],
                core_axis_name=("c", "s"),
                dimension_semantics=(pltpu.PARALLEL,),
            )(t_hbm, out_hbm)

        run(ut_hbm, up_hbm, UA_BLKS)
        run(dt_hbm, dp_hbm, nd // REPACK_W)

    return repack_kernel(user_tT, diner_tT)


def _tc_repack_partB(tT):
    n = tT.shape[1]
    rows_b = (n - UA_BLKS * REPACK_W) // PACK
    grid = ((n - UA_BLKS * REPACK_W + TCB_W - 1) // TCB_W,)
    return pl.pallas_call(
        _repack_body,
        grid=grid,
        in_specs=[pl.BlockSpec((ED, TCB_W), lambda i: (0, i + UB_START_BLK))],
        out_specs=pl.BlockSpec((TCB_W // PACK, 128), lambda i: (i, 0)),
        out_shape=jax.ShapeDtypeStruct((rows_b, 128), jnp.float32),
    )(tT)


def _repack_body(in_ref, out_ref):
    x = in_ref[...]
    y = jnp.transpose(x)
    y3 = y.reshape(-1, PACK, ED)
    out_ref[...] = jnp.concatenate([y3[:, j, :] for j in range(PACK)],
                                   axis=1)


def _tail_body(t_ref, packed_ref, out_ref):
    del packed_ref
    x = t_ref[...]
    y = jnp.transpose(x)
    y3 = y.reshape(-1, PACK, ED)
    out_ref[...] = jnp.concatenate([y3[:, j, :] for j in range(PACK)],
                                   axis=1)


def _tc_tail_repack(tT, packed):
    n = tT.shape[1]
    k = n // REPACK_W
    rows = packed.shape[0]
    return pl.pallas_call(
        _tail_body,
        grid=(1,),
        in_specs=[
            pl.BlockSpec((ED, REPACK_W), lambda i: (0, k)),
            pl.BlockSpec(memory_space=pltpu.MemorySpace.HBM),
        ],
        out_specs=pl.BlockSpec((REPACK_W // PACK, 128), lambda i: (k, 0)),
        out_shape=jax.ShapeDtypeStruct((rows, 128), jnp.float32),
        input_output_aliases={1: 0},
    )(tT, packed)


def _sc_gather(tab_a, idx_a, tab_b, idx_b, tab_d, idx_d):
    mesh = plsc.VectorSubcoreMesh(core_axis_name="c", subcore_axis_name="s")

    @functools.partial(
        pl.kernel,
        out_type=(
            jax.ShapeDtypeStruct((B, 128), jnp.float32),
            jax.ShapeDtypeStruct((B, 128), jnp.float32),
            jax.ShapeDtypeStruct((B, 128), jnp.float32),
        ),
        mesh=mesh,
    )
    def gather_kernel(ta_hbm, ia_hbm, tb_hbm, ib_hbm, td_hbm, id_hbm,
                      oa_hbm, ob_hbm, od_hbm):
        def make_body(table_hbm):
            def body(i_vmem, o_vmem):
                pltpu.sync_copy(table_hbm.at[i_vmem.at[0]], o_vmem)
            return body

        def run(table_hbm, idx_hbm, out_hbm):
            pltpu.emit_pipeline(
                make_body(table_hbm),
                grid=(B // GATHER_W,),
                in_specs=[pl.BlockSpec((1, GATHER_W), lambda i: (0, i))],
                out_specs=[pl.BlockSpec((GATHER_W, 128), lambda i: (i, 0))],
                core_axis_name=("c", "s"),
                dimension_semantics=(pltpu.PARALLEL,),
            )(idx_hbm, out_hbm)

        run(ta_hbm, ia_hbm, oa_hbm)
        run(tb_hbm, ib_hbm, ob_hbm)
        run(td_hbm, id_hbm, od_hbm)

    return gather_kernel(tab_a, idx_a, tab_b, idx_b, tab_d, idx_d)


def _mlp_body(uepa, uepb, dep, mods, f, bkt, kpat, expand, patt, Wcat,
              Wu, Wd, Wf, b1, fn_g, fn_b, g1, bb1, W2, b2, g2, bb2,
              W3, b3, out):
    mc = jnp.transpose(mods[...])
    umod, dmod, sel_a = mc[:, 0:1], mc[:, 1:2], mc[:, 2:3]
    mu = (umod == kpat[...]).astype(jnp.bfloat16)
    md = (dmod == kpat[...]).astype(jnp.bfloat16)
    ue_pack = uepa[...] * sel_a + uepb[...] * (1.0 - sel_a)
    pu = ue_pack.astype(jnp.bfloat16) * mu
    pd = dep[...].astype(jnp.bfloat16) * md

    fx = f[...]
    m = jnp.mean(fx, axis=-1, keepdims=True)
    v = jnp.mean((fx - m) ** 2, axis=-1, keepdims=True)
    fln = (fx - m) * jax.lax.rsqrt(v + 1e-5) * fn_g[...] + fn_b[...]

    rep = jnp.dot(bkt[...], expand[...], preferred_element_type=jnp.float32)
    mh = (rep == patt[...]).astype(jnp.bfloat16)

    h = jnp.dot(mh, Wcat[...], preferred_element_type=jnp.float32)
    h = h + jnp.dot(pu, Wu[...], preferred_element_type=jnp.float32)
    h = h + jnp.dot(pd, Wd[...], preferred_element_type=jnp.float32)
    h = h + jnp.dot(fln.astype(jnp.bfloat16), Wf[...],
                    preferred_element_type=jnp.float32)
    h = h + b1[...]
    m = jnp.mean(h, axis=-1, keepdims=True)
    v = jnp.mean((h - m) ** 2, axis=-1, keepdims=True)
    h = (h - m) * jax.lax.rsqrt(v + 1e-5) * g1[...] + bb1[...]
    h = jnp.maximum(h, 0.0).astype(jnp.bfloat16)

    h = jnp.dot(h, W2[...], preferred_element_type=jnp.float32) + b2[...]
    m = jnp.mean(h, axis=-1, keepdims=True)
    v = jnp.mean((h - m) ** 2, axis=-1, keepdims=True)
    h = (h - m) * jax.lax.rsqrt(v + 1e-5) * g2[...] + bb2[...]
    h = jnp.maximum(h, 0.0).astype(jnp.bfloat16)

    o = jnp.dot(h, W3[...], preferred_element_type=jnp.float32) + b3[...]
    out[...] = jax.nn.sigmoid(o)


def _tc_mlp(uepa, uepb, dep, mods, features, bkt, kpat, expand, patt, Wcat,
            Wu, Wd, Wf, b1, fn_g, fn_b, ln1_g, ln1_b, W2, b2, ln2_g, ln2_b,
            W3, b3):
    grid = (B // MLP_BLK,)

    def row_spec(cols):
        return pl.BlockSpec((MLP_BLK, cols), lambda i: (i, 0))

    def full_spec(a):
        return pl.BlockSpec(a.shape, lambda i: (0,) * a.ndim)

    out = pl.pallas_call(
        _mlp_body,
        grid=grid,
        in_specs=[
            row_spec(128), row_spec(128), row_spec(128),
            pl.BlockSpec((3, MLP_BLK), lambda i: (0, i)),
            row_spec(NF), row_spec(NC),
            full_spec(kpat), full_spec(expand), full_spec(patt),
            full_spec(Wcat), full_spec(Wu), full_spec(Wd), full_spec(Wf),
            full_spec(b1), full_spec(fn_g), full_spec(fn_b),
            full_spec(ln1_g), full_spec(ln1_b),
            full_spec(W2), full_spec(b2), full_spec(ln2_g), full_spec(ln2_b),
            full_spec(W3), full_spec(b3),
        ],
        out_specs=pl.BlockSpec((MLP_BLK, 1), lambda i: (i, 0)),
        out_shape=jax.ShapeDtypeStruct((B, 1), jnp.float32),
    )(uepa, uepb, dep, mods, features, bkt, kpat, expand, patt, Wcat,
      Wu, Wd, Wf, b1, fn_g, fn_b, ln1_g, ln1_b, W2, b2, ln2_g, ln2_b,
      W3, b3)
    return out[:, 0]


def kernel(user_idx, diner_idx, features, categorical_bucket_idx,
           user_table, diner_table, cat_tables, fn_g, fn_b, W1, b1,
           ln1_g, ln1_b, W2, b2, ln2_g, ln2_b, W3, b3):
    uidx = user_idx.astype(jnp.int32)
    didx = diner_idx.astype(jnp.int32)

    upack_a, diner_packed = _sc_repack(user_table.T, diner_table.T)
    upack_b = _tc_repack_partB(user_table.T)
    diner_packed = _tc_tail_repack(diner_table.T, diner_packed)

    upk = uidx // PACK
    in_a = upk < UA_ROWS
    upk_a = jnp.where(in_a, upk, upk % UA_ROWS)
    upk_b = jnp.where(in_a, upk % jnp.int32(upack_b.shape[0]), upk - UA_ROWS)
    uepa, uepb, dep = _sc_gather(upack_a, upk_a.reshape(1, B),
                                 upack_b, upk_b.reshape(1, B),
                                 diner_packed, (didx // PACK).reshape(1, B))
    mods = jnp.stack([
        (uidx % PACK).astype(jnp.float32),
        (didx % PACK).astype(jnp.float32),
        (upk < UA_ROWS).astype(jnp.float32),
    ])

    W1c = W1[2 * ED + NF:].reshape(NC, CD, H1)
    Wcat = jnp.einsum("cbd,cdh->cbh", cat_tables, W1c,
                      preferred_element_type=jnp.float32)
    Wcat = Wcat.reshape(NCLS, H1).astype(jnp.bfloat16)

    cls = jnp.arange(NCLS, dtype=jnp.int32)
    expand = (cls[None, :] // NB == jnp.arange(NC, dtype=jnp.int32)[:, None])
    expand = expand.astype(jnp.bfloat16)
    patt = (cls % NB).astype(jnp.float32).reshape(1, NCLS)
    bkt = categorical_bucket_idx.astype(jnp.bfloat16)
    kpat = (jnp.arange(128, dtype=jnp.int32) // ED).astype(
        jnp.float32).reshape(1, 128)

    Wb = W1.astype(jnp.bfloat16)
    Wu = jnp.tile(Wb[:ED], (PACK, 1))
    Wd = jnp.tile(Wb[ED:2 * ED], (PACK, 1))
    Wf = Wb[2 * ED:2 * ED + NF]

    out = _tc_mlp(uepa, uepb, dep, mods,
                  features, bkt, kpat, expand, patt, Wcat, Wu, Wd, Wf,
                  b1.reshape(1, H1),
                  fn_g.reshape(1, NF), fn_b.reshape(1, NF),
                  ln1_g.reshape(1, H1), ln1_b.reshape(1, H1),
                  W2.astype(jnp.bfloat16), b2.reshape(1, H2),
                  ln2_g.reshape(1, H2), ln2_b.reshape(1, H2),
                  W3.astype(jnp.bfloat16), b3.reshape(1, 1))
    return out


# speedup vs baseline: 3.7106x; 1.0718x over previous
import dataclasses
import functools

import jax
import jax.numpy as jnp
from jax.experimental import pallas as pl
from jax.experimental.pallas import tpu as pltpu
from jax.experimental.pallas import tpu_sc as plsc

B = 16384
ED = 16
NF = 13
NC, NB, CD = 26, 20, 8
NCLS = NC * NB
H1, H2 = 256, 128
PACK = 128 // ED

GATHER_W = 128
MLP_BLK = 1024


REPACK_W = 1024
UA_BLKS = 544
UA_ROWS = UA_BLKS * REPACK_W // PACK
TCB_W = 16384
UB_START_BLK = UA_BLKS * REPACK_W // TCB_W


def _sc_repack(user_tT, diner_tT):
    mesh = plsc.VectorSubcoreMesh(core_axis_name="c", subcore_axis_name="s")
    nu = user_tT.shape[1]
    nd = diner_tT.shape[1]
    cp = pltpu.CompilerParams()
    if "needs_layout_passes" in pltpu.CompilerParams.__dataclass_fields__:
        cp = dataclasses.replace(cp, needs_layout_passes=False)

    @functools.partial(
        pl.kernel,
        out_type=(
            jax.ShapeDtypeStruct((UA_ROWS, 128), jnp.float32),
            jax.ShapeDtypeStruct((nd // PACK, 128), jnp.float32),
        ),
        mesh=mesh,
        compiler_params=cp,
    )
    def repack_kernel(ut_hbm, dt_hbm, up_hbm, dp_hbm):
        def body(in_vmem, out_vmem):
            d_vec = jax.lax.iota(jnp.int32, ED)

            @plsc.parallel_loop(0, REPACK_W // PACK)
            def _(p):
                base = jnp.full((ED,), p * PACK, jnp.int32)
                row = jnp.full((ED,), p, jnp.int32)
                for j in range(PACK):
                    v = plsc.load_gather(in_vmem, [d_vec, base + j])
                    plsc.store_scatter(out_vmem, [row, d_vec + j * ED], v)

        def run(t_hbm, out_hbm, nblk):
            pltpu.emit_pipeline(
                body,
                grid=(nblk,),
                in_specs=[pl.BlockSpec((ED, REPACK_W), lambda i: (0, i))],
                out_specs=[pl.BlockSpec((REPACK_W // PACK, 128),
                                        lambda i: (i, 0))],
                core_axis_name=("c", "s"),
                dimension_semantics=(pltpu.PARALLEL,),
            )(t_hbm, out_hbm)

        run(ut_hbm, up_hbm, UA_BLKS)
        run(dt_hbm, dp_hbm, nd // REPACK_W)

    return repack_kernel(user_tT, diner_tT)


def _tc_repack_partB(tT):
    n = tT.shape[1]
    rows_b = (n - UA_BLKS * REPACK_W) // PACK
    grid = ((n - UA_BLKS * REPACK_W + TCB_W - 1) // TCB_W,)
    return pl.pallas_call(
        _repack_body,
        grid=grid,
        in_specs=[pl.BlockSpec((ED, TCB_W), lambda i: (0, i + UB_START_BLK))],
        out_specs=pl.BlockSpec((TCB_W // PACK, 128), lambda i: (i, 0)),
        out_shape=jax.ShapeDtypeStruct((rows_b, 128), jnp.float32),
    )(tT)


def _repack_body(in_ref, out_ref):
    x = in_ref[...]
    y = jnp.transpose(x)
    y3 = y.reshape(-1, PACK, ED)
    out_ref[...] = jnp.concatenate([y3[:, j, :] for j in range(PACK)],
                                   axis=1)


def _tail_body(t_ref, packed_ref, out_ref):
    del packed_ref
    x = t_ref[...]
    y = jnp.transpose(x)
    y3 = y.reshape(-1, PACK, ED)
    out_ref[...] = jnp.concatenate([y3[:, j, :] for j in range(PACK)],
                                   axis=1)


def _tc_tail_repack(tT, packed):
    n = tT.shape[1]
    k = n // REPACK_W
    rows = packed.shape[0]
    return pl.pallas_call(
        _tail_body,
        grid=(1,),
        in_specs=[
            pl.BlockSpec((ED, REPACK_W), lambda i: (0, k)),
            pl.BlockSpec(memory_space=pltpu.MemorySpace.HBM),
        ],
        out_specs=pl.BlockSpec((REPACK_W // PACK, 128), lambda i: (k, 0)),
        out_shape=jax.ShapeDtypeStruct((rows, 128), jnp.float32),
        input_output_aliases={1: 0},
    )(tT, packed)


def _sc_gather(tab_a, idx_a, tab_b, idx_b, tab_d, idx_d):
    mesh = plsc.VectorSubcoreMesh(core_axis_name="c", subcore_axis_name="s")

    @functools.partial(
        pl.kernel,
        out_type=(
            jax.ShapeDtypeStruct((B, 128), jnp.float32),
            jax.ShapeDtypeStruct((B, 128), jnp.float32),
            jax.ShapeDtypeStruct((B, 128), jnp.float32),
        ),
        mesh=mesh,
    )
    def gather_kernel(ta_hbm, ia_hbm, tb_hbm, ib_hbm, td_hbm, id_hbm,
                      oa_hbm, ob_hbm, od_hbm):
        def make_body(table_hbm):
            def body(i_vmem, o_vmem):
                pltpu.sync_copy(table_hbm.at[i_vmem.at[0]], o_vmem)
            return body

        def run(table_hbm, idx_hbm, out_hbm):
            pltpu.emit_pipeline(
                make_body(table_hbm),
                grid=(B // GATHER_W,),
                in_specs=[pl.BlockSpec((1, GATHER_W), lambda i: (0, i))],
                out_specs=[pl.BlockSpec((GATHER_W, 128), lambda i: (i, 0))],
                core_axis_name=("c", "s"),
                dimension_semantics=(pltpu.PARALLEL,),
            )(idx_hbm, out_hbm)

        run(ta_hbm, ia_hbm, oa_hbm)
        run(tb_hbm, ib_hbm, ob_hbm)
        run(td_hbm, id_hbm, od_hbm)

    return gather_kernel(tab_a, idx_a, tab_b, idx_b, tab_d, idx_d)


def _mlp_body(uepa, uepb, dep, mods, f, bkt, kpat, expand, patt, Wcat,
              Wu, Wd, Wf, b1, fn_g, fn_b, g1, bb1, W2, b2, g2, bb2,
              W3, b3, out):
    mc = jnp.transpose(mods[...])
    umod, dmod, sel_a = mc[:, 0:1], mc[:, 1:2], mc[:, 2:3]
    mu = (umod == kpat[...]).astype(jnp.bfloat16)
    md = (dmod == kpat[...]).astype(jnp.bfloat16)
    ue_pack = uepa[...] * sel_a + uepb[...] * (1.0 - sel_a)
    pu = ue_pack.astype(jnp.bfloat16) * mu
    pd = dep[...].astype(jnp.bfloat16) * md

    fx = f[...]
    m = jnp.mean(fx, axis=-1, keepdims=True)
    v = jnp.mean((fx - m) ** 2, axis=-1, keepdims=True)
    fln = (fx - m) * jax.lax.rsqrt(v + 1e-5) * fn_g[...] + fn_b[...]

    rep = jnp.dot(bkt[...], expand[...], preferred_element_type=jnp.float32)
    mh = (rep == patt[...]).astype(jnp.bfloat16)

    h = jnp.dot(mh, Wcat[...], preferred_element_type=jnp.float32)
    h = h + jnp.dot(pu, Wu[...], preferred_element_type=jnp.float32)
    h = h + jnp.dot(pd, Wd[...], preferred_element_type=jnp.float32)
    h = h + jnp.dot(fln.astype(jnp.bfloat16), Wf[...],
                    preferred_element_type=jnp.float32)
    h = h + b1[...]
    m = jnp.mean(h, axis=-1, keepdims=True)
    v = jnp.mean((h - m) ** 2, axis=-1, keepdims=True)
    h = (h - m) * jax.lax.rsqrt(v + 1e-5) * g1[...] + bb1[...]
    h = jnp.maximum(h, 0.0).astype(jnp.bfloat16)

    h = jnp.dot(h, W2[...], preferred_element_type=jnp.float32) + b2[...]
    m = jnp.mean(h, axis=-1, keepdims=True)
    v = jnp.mean((h - m) ** 2, axis=-1, keepdims=True)
    h = (h - m) * jax.lax.rsqrt(v + 1e-5) * g2[...] + bb2[...]
    h = jnp.maximum(h, 0.0).astype(jnp.bfloat16)

    o = jnp.dot(h, W3[...], preferred_element_type=jnp.float32) + b3[...]
    out[...] = jax.nn.sigmoid(o)


def _tc_mlp(uepa, uepb, dep, mods, features, bkt, kpat, expand, patt, Wcat,
            Wu, Wd, Wf, b1, fn_g, fn_b, ln1_g, ln1_b, W2, b2, ln2_g, ln2_b,
            W3, b3):
    grid = (B // MLP_BLK,)

    def row_spec(cols):
        return pl.BlockSpec((MLP_BLK, cols), lambda i: (i, 0))

    def full_spec(a):
        return pl.BlockSpec(a.shape, lambda i: (0,) * a.ndim)

    out = pl.pallas_call(
        _mlp_body,
        grid=grid,
        in_specs=[
            row_spec(128), row_spec(128), row_spec(128),
            pl.BlockSpec((3, MLP_BLK), lambda i: (0, i)),
            row_spec(NF), row_spec(NC),
            full_spec(kpat), full_spec(expand), full_spec(patt),
            full_spec(Wcat), full_spec(Wu), full_spec(Wd), full_spec(Wf),
            full_spec(b1), full_spec(fn_g), full_spec(fn_b),
            full_spec(ln1_g), full_spec(ln1_b),
            full_spec(W2), full_spec(b2), full_spec(ln2_g), full_spec(ln2_b),
            full_spec(W3), full_spec(b3),
        ],
        out_specs=pl.BlockSpec((MLP_BLK, 1), lambda i: (i, 0)),
        out_shape=jax.ShapeDtypeStruct((B, 1), jnp.float32),
    )(uepa, uepb, dep, mods, features, bkt, kpat, expand, patt, Wcat,
      Wu, Wd, Wf, b1, fn_g, fn_b, ln1_g, ln1_b, W2, b2, ln2_g, ln2_b,
      W3, b3)
    return out[:, 0]


def kernel(user_idx, diner_idx, features, categorical_bucket_idx,
           user_table, diner_table, cat_tables, fn_g, fn_b, W1, b1,
           ln1_g, ln1_b, W2, b2, ln2_g, ln2_b, W3, b3):
    uidx = user_idx.astype(jnp.int32)
    didx = diner_idx.astype(jnp.int32)

    upack_a, diner_packed = _sc_repack(user_table.T, diner_table.T)
    upack_b = _tc_repack_partB(user_table.T)
    diner_packed = _tc_tail_repack(diner_table.T, diner_packed)

    upk = uidx // PACK
    in_a = upk < UA_ROWS
    upk_a = jnp.where(in_a, upk, upk % UA_ROWS)
    upk_b = jnp.where(in_a, upk % jnp.int32(upack_b.shape[0]), upk - UA_ROWS)
    uepa, uepb, dep = _sc_gather(upack_a, upk_a.reshape(1, B),
                                 upack_b, upk_b.reshape(1, B),
                                 diner_packed, (didx // PACK).reshape(1, B))
    mods = jnp.stack([
        (uidx % PACK).astype(jnp.float32),
        (didx % PACK).astype(jnp.float32),
        (upk < UA_ROWS).astype(jnp.float32),
    ])

    W1c = W1[2 * ED + NF:].reshape(NC, CD, H1)
    Wcat = jnp.einsum("cbd,cdh->cbh", cat_tables, W1c,
                      preferred_element_type=jnp.float32)
    Wcat = Wcat.reshape(NCLS, H1).astype(jnp.bfloat16)

    cls = jnp.arange(NCLS, dtype=jnp.int32)
    expand = (cls[None, :] // NB == jnp.arange(NC, dtype=jnp.int32)[:, None])
    expand = expand.astype(jnp.bfloat16)
    patt = (cls % NB).astype(jnp.float32).reshape(1, NCLS)
    bkt = categorical_bucket_idx.astype(jnp.bfloat16)
    kpat = (jnp.arange(128, dtype=jnp.int32) // ED).astype(
        jnp.float32).reshape(1, 128)

    Wb = W1.astype(jnp.bfloat16)
    Wu = jnp.tile(Wb[:ED], (PACK, 1))
    Wd = jnp.tile(Wb[ED:2 * ED], (PACK, 1))
    Wf = Wb[2 * ED:2 * ED + NF]

    out = _tc_mlp(uepa, uepb, dep, mods,
                  features, bkt, kpat, expand, patt, Wcat, Wu, Wd, Wf,
                  b1.reshape(1, H1),
                  fn_g.reshape(1, NF), fn_b.reshape(1, NF),
                  ln1_g.reshape(1, H1), ln1_b.reshape(1, H1),
                  W2.astype(jnp.bfloat16), b2.reshape(1, H2),
                  ln2_g.reshape(1, H2), ln2_b.reshape(1, H2),
                  W3.astype(jnp.bfloat16), b3.reshape(1, 1))
    return out


# speedup vs baseline: 3.7251x; 1.0039x over previous
import dataclasses
import functools

import jax
import jax.numpy as jnp
from jax.experimental import pallas as pl
from jax.experimental.pallas import tpu as pltpu
from jax.experimental.pallas import tpu_sc as plsc

B = 16384
ED = 16
NF = 13
NC, NB, CD = 26, 20, 8
NCLS = NC * NB
H1, H2 = 256, 128
PACK = 128 // ED

GATHER_W = 128
MLP_BLK = 2048


REPACK_W = 1024
UA_BLKS = 544
UA_ROWS = UA_BLKS * REPACK_W // PACK
TCB_W = 16384
UB_START_BLK = UA_BLKS * REPACK_W // TCB_W


def _sc_repack(user_tT, diner_tT):
    mesh = plsc.VectorSubcoreMesh(core_axis_name="c", subcore_axis_name="s")
    nu = user_tT.shape[1]
    nd = diner_tT.shape[1]
    cp = pltpu.CompilerParams()
    if "needs_layout_passes" in pltpu.CompilerParams.__dataclass_fields__:
        cp = dataclasses.replace(cp, needs_layout_passes=False)

    @functools.partial(
        pl.kernel,
        out_type=(
            jax.ShapeDtypeStruct((UA_ROWS, 128), jnp.float32),
            jax.ShapeDtypeStruct((nd // PACK, 128), jnp.float32),
        ),
        mesh=mesh,
        compiler_params=cp,
    )
    def repack_kernel(ut_hbm, dt_hbm, up_hbm, dp_hbm):
        def body(in_vmem, out_vmem):
            d_vec = jax.lax.iota(jnp.int32, ED)

            @plsc.parallel_loop(0, REPACK_W // PACK)
            def _(p):
                base = jnp.full((ED,), p * PACK, jnp.int32)
                row = jnp.full((ED,), p, jnp.int32)
                for j in range(PACK):
                    v = plsc.load_gather(in_vmem, [d_vec, base + j])
                    plsc.store_scatter(out_vmem, [row, d_vec + j * ED], v)

        def run(t_hbm, out_hbm, nblk):
            pltpu.emit_pipeline(
                body,
                grid=(nblk,),
                in_specs=[pl.BlockSpec((ED, REPACK_W), lambda i: (0, i))],
                out_specs=[pl.BlockSpec((REPACK_W // PACK, 128),
                                        lambda i: (i, 0))],
                core_axis_name=("c", "s"),
                dimension_semantics=(pltpu.PARALLEL,),
            )(t_hbm, out_hbm)

        run(ut_hbm, up_hbm, UA_BLKS)
        run(dt_hbm, dp_hbm, nd // REPACK_W)

    return repack_kernel(user_tT, diner_tT)


def _tc_repack_partB(tT):
    n = tT.shape[1]
    rows_b = (n - UA_BLKS * REPACK_W) // PACK
    grid = ((n - UA_BLKS * REPACK_W + TCB_W - 1) // TCB_W,)
    return pl.pallas_call(
        _repack_body,
        grid=grid,
        in_specs=[pl.BlockSpec((ED, TCB_W), lambda i: (0, i + UB_START_BLK))],
        out_specs=pl.BlockSpec((TCB_W // PACK, 128), lambda i: (i, 0)),
        out_shape=jax.ShapeDtypeStruct((rows_b, 128), jnp.float32),
    )(tT)


def _repack_body(in_ref, out_ref):
    x = in_ref[...]
    y = jnp.transpose(x)
    y3 = y.reshape(-1, PACK, ED)
    out_ref[...] = jnp.concatenate([y3[:, j, :] for j in range(PACK)],
                                   axis=1)


def _tail_body(t_ref, packed_ref, out_ref):
    del packed_ref
    x = t_ref[...]
    y = jnp.transpose(x)
    y3 = y.reshape(-1, PACK, ED)
    out_ref[...] = jnp.concatenate([y3[:, j, :] for j in range(PACK)],
                                   axis=1)


def _tc_tail_repack(tT, packed):
    n = tT.shape[1]
    k = n // REPACK_W
    rows = packed.shape[0]
    return pl.pallas_call(
        _tail_body,
        grid=(1,),
        in_specs=[
            pl.BlockSpec((ED, REPACK_W), lambda i: (0, k)),
            pl.BlockSpec(memory_space=pltpu.MemorySpace.HBM),
        ],
        out_specs=pl.BlockSpec((REPACK_W // PACK, 128), lambda i: (k, 0)),
        out_shape=jax.ShapeDtypeStruct((rows, 128), jnp.float32),
        input_output_aliases={1: 0},
    )(tT, packed)


def _sc_gather(tab_a, idx_a, tab_b, idx_b, tab_d, idx_d):
    mesh = plsc.VectorSubcoreMesh(core_axis_name="c", subcore_axis_name="s")

    @functools.partial(
        pl.kernel,
        out_type=(
            jax.ShapeDtypeStruct((B, 128), jnp.float32),
            jax.ShapeDtypeStruct((B, 128), jnp.float32),
            jax.ShapeDtypeStruct((B, 128), jnp.float32),
        ),
        mesh=mesh,
    )
    def gather_kernel(ta_hbm, ia_hbm, tb_hbm, ib_hbm, td_hbm, id_hbm,
                      oa_hbm, ob_hbm, od_hbm):
        def make_body(table_hbm):
            def body(i_vmem, o_vmem):
                pltpu.sync_copy(table_hbm.at[i_vmem.at[0]], o_vmem)
            return body

        def run(table_hbm, idx_hbm, out_hbm):
            pltpu.emit_pipeline(
                make_body(table_hbm),
                grid=(B // GATHER_W,),
                in_specs=[pl.BlockSpec((1, GATHER_W), lambda i: (0, i))],
                out_specs=[pl.BlockSpec((GATHER_W, 128), lambda i: (i, 0))],
                core_axis_name=("c", "s"),
                dimension_semantics=(pltpu.PARALLEL,),
            )(idx_hbm, out_hbm)

        run(ta_hbm, ia_hbm, oa_hbm)
        run(tb_hbm, ib_hbm, ob_hbm)
        run(td_hbm, id_hbm, od_hbm)

    return gather_kernel(tab_a, idx_a, tab_b, idx_b, tab_d, idx_d)


def _mlp_body(uepa, uepb, dep, mods, f, bkt, kpat, expand, patt, Wcat,
              Wu, Wd, Wf, b1, fn_g, fn_b, g1, bb1, W2, b2, g2, bb2,
              W3, b3, out):
    mc = jnp.transpose(mods[...])
    umod, dmod, sel_a = mc[:, 0:1], mc[:, 1:2], mc[:, 2:3]
    mu = (umod == kpat[...]).astype(jnp.bfloat16)
    md = (dmod == kpat[...]).astype(jnp.bfloat16)
    sa = sel_a.astype(jnp.float32)
    ue_pack = uepa[...] * sa + uepb[...] * (1.0 - sa)
    pu = ue_pack.astype(jnp.bfloat16) * mu
    pd = dep[...].astype(jnp.bfloat16) * md

    fx = f[...]
    m = jnp.mean(fx, axis=-1, keepdims=True)
    v = jnp.mean((fx - m) ** 2, axis=-1, keepdims=True)
    fln = (fx - m) * jax.lax.rsqrt(v + 1e-5) * fn_g[...] + fn_b[...]

    rep = jnp.dot(bkt[...], expand[...], preferred_element_type=jnp.float32)
    mh = (rep == patt[...]).astype(jnp.bfloat16)

    h = jnp.dot(mh, Wcat[...], preferred_element_type=jnp.float32)
    h = h + jnp.dot(pu, Wu[...], preferred_element_type=jnp.float32)
    h = h + jnp.dot(pd, Wd[...], preferred_element_type=jnp.float32)
    h = h + jnp.dot(fln.astype(jnp.bfloat16), Wf[...],
                    preferred_element_type=jnp.float32)
    h = h + b1[...]
    m = jnp.mean(h, axis=-1, keepdims=True)
    v = jnp.mean((h - m) ** 2, axis=-1, keepdims=True)
    h = (h - m) * jax.lax.rsqrt(v + 1e-5) * g1[...] + bb1[...]
    h = jnp.maximum(h, 0.0).astype(jnp.bfloat16)

    h = jnp.dot(h, W2[...], preferred_element_type=jnp.float32) + b2[...]
    m = jnp.mean(h, axis=-1, keepdims=True)
    v = jnp.mean((h - m) ** 2, axis=-1, keepdims=True)
    h = (h - m) * jax.lax.rsqrt(v + 1e-5) * g2[...] + bb2[...]
    h = jnp.maximum(h, 0.0).astype(jnp.bfloat16)

    o = jnp.dot(h, W3[...], preferred_element_type=jnp.float32) + b3[...]
    out[...] = jax.nn.sigmoid(o)


def _tc_mlp(uepa, uepb, dep, mods, features, bkt, kpat, expand, patt, Wcat,
            Wu, Wd, Wf, b1, fn_g, fn_b, ln1_g, ln1_b, W2, b2, ln2_g, ln2_b,
            W3, b3):
    grid = (B // MLP_BLK,)

    def row_spec(cols):
        return pl.BlockSpec((MLP_BLK, cols), lambda i: (i, 0))

    def full_spec(a):
        return pl.BlockSpec(a.shape, lambda i: (0,) * a.ndim)

    out = pl.pallas_call(
        _mlp_body,
        grid=grid,
        in_specs=[
            row_spec(128), row_spec(128), row_spec(128),
            pl.BlockSpec((3, MLP_BLK), lambda i: (0, i)),
            row_spec(NF), row_spec(NC),
            full_spec(kpat), full_spec(expand), full_spec(patt),
            full_spec(Wcat), full_spec(Wu), full_spec(Wd), full_spec(Wf),
            full_spec(b1), full_spec(fn_g), full_spec(fn_b),
            full_spec(ln1_g), full_spec(ln1_b),
            full_spec(W2), full_spec(b2), full_spec(ln2_g), full_spec(ln2_b),
            full_spec(W3), full_spec(b3),
        ],
        out_specs=pl.BlockSpec((MLP_BLK, 1), lambda i: (i, 0)),
        out_shape=jax.ShapeDtypeStruct((B, 1), jnp.float32),
    )(uepa, uepb, dep, mods, features, bkt, kpat, expand, patt, Wcat,
      Wu, Wd, Wf, b1, fn_g, fn_b, ln1_g, ln1_b, W2, b2, ln2_g, ln2_b,
      W3, b3)
    return out[:, 0]


def kernel(user_idx, diner_idx, features, categorical_bucket_idx,
           user_table, diner_table, cat_tables, fn_g, fn_b, W1, b1,
           ln1_g, ln1_b, W2, b2, ln2_g, ln2_b, W3, b3):
    uidx = user_idx.astype(jnp.int32)
    didx = diner_idx.astype(jnp.int32)

    upack_a, diner_packed = _sc_repack(user_table.T, diner_table.T)
    upack_b = _tc_repack_partB(user_table.T)
    diner_packed = _tc_tail_repack(diner_table.T, diner_packed)

    upk = uidx // PACK
    in_a = upk < UA_ROWS
    upk_a = jnp.where(in_a, upk, upk % UA_ROWS)
    upk_b = jnp.where(in_a, upk % jnp.int32(upack_b.shape[0]), upk - UA_ROWS)
    uepa, uepb, dep = _sc_gather(upack_a, upk_a.reshape(1, B),
                                 upack_b, upk_b.reshape(1, B),
                                 diner_packed, (didx // PACK).reshape(1, B))
    mods = jnp.stack([
        (uidx % PACK).astype(jnp.bfloat16),
        (didx % PACK).astype(jnp.bfloat16),
        (upk < UA_ROWS).astype(jnp.bfloat16),
    ])

    W1c = W1[2 * ED + NF:].reshape(NC, CD, H1)
    Wcat = jnp.einsum("cbd,cdh->cbh", cat_tables, W1c,
                      preferred_element_type=jnp.float32)
    Wcat = Wcat.reshape(NCLS, H1).astype(jnp.bfloat16)

    cls = jnp.arange(NCLS, dtype=jnp.int32)
    expand = (cls[None, :] // NB == jnp.arange(NC, dtype=jnp.int32)[:, None])
    expand = expand.astype(jnp.bfloat16)
    patt = (cls % NB).astype(jnp.float32).reshape(1, NCLS)
    bkt = categorical_bucket_idx.astype(jnp.bfloat16)
    kpat = (jnp.arange(128, dtype=jnp.int32) // ED).astype(
        jnp.bfloat16).reshape(1, 128)

    Wb = W1.astype(jnp.bfloat16)
    Wu = jnp.tile(Wb[:ED], (PACK, 1))
    Wd = jnp.tile(Wb[ED:2 * ED], (PACK, 1))
    Wf = Wb[2 * ED:2 * ED + NF]

    out = _tc_mlp(uepa, uepb, dep, mods,
                  features, bkt, kpat, expand, patt, Wcat, Wu, Wd, Wf,
                  b1.reshape(1, H1),
                  fn_g.reshape(1, NF), fn_b.reshape(1, NF),
                  ln1_g.reshape(1, H1), ln1_b.reshape(1, H1),
                  W2.astype(jnp.bfloat16), b2.reshape(1, H2),
                  ln2_g.reshape(1, H2), ln2_b.reshape(1, H2),
                  W3.astype(jnp.bfloat16), b3.reshape(1, 1))
    return out
